# baseline (speedup 1.0000x reference)
import functools

import jax
import jax.numpy as jnp
from jax import lax
from jax.experimental import pallas as pl
from jax.experimental.pallas import tpu as pltpu
from jax.experimental.pallas import tpu_sc as plsc

N = 10000
D = 128
E = 320000

NC = 2
NS = 16
NW = NC * NS
K = 128
FULL_CHUNKS = E // K
CHUNKS = 80
NP = 10240
ROWS_PER_TILE = NP // NS
DUMMY = N
ZR = 8


def _sc_scatter(ei, a0, a1, wvec, x):
    mesh = plsc.VectorSubcoreMesh(
        core_axis_name="c", subcore_axis_name="s", num_cores=NC, num_subcores=NS
    )

    @functools.partial(
        pl.kernel,
        out_type=jax.ShapeDtypeStruct((NC * NP, D), jnp.float32),
        mesh=mesh,
        scratch_types=[
            pltpu.VMEM((K,), jnp.int32),
            pltpu.VMEM((K,), jnp.int32),
            pltpu.VMEM((K,), jnp.int32),
            pltpu.VMEM((K,), jnp.int32),
            pltpu.VMEM((K,), jnp.float32),
            pltpu.VMEM((K,), jnp.float32),
            pltpu.VMEM((K,), jnp.float32),
            pltpu.VMEM((K,), jnp.float32),
            pltpu.VMEM((2, 16), jnp.float32),
            pltpu.VMEM((K, D), jnp.float32),
            pltpu.VMEM((K, D), jnp.float32),
            pltpu.VMEM((K,), jnp.int32),
            pltpu.VMEM((K,), jnp.int32),
            pltpu.VMEM((ZR, D), jnp.float32),
            pltpu.VMEM_SHARED((NP, D), jnp.float32),
            pltpu.SemaphoreType.DMA,
            pltpu.SemaphoreType.DMA,
            pltpu.SemaphoreType.DMA,
            pltpu.SemaphoreType.DMA,
            pltpu.SemaphoreType.DMA,
            pltpu.SemaphoreType.DMA,
            pltpu.SemaphoreType.DMA,
            pltpu.SemaphoreType.DMA,
            pltpu.SemaphoreType.DMA,
            pltpu.SemaphoreType.DMA,
        ],
    )
    def sc_kernel(ei_hbm, a0_hbm, a1_hbm, w_hbm, x_hbm, out_hbm,
                  s0, s1, d0, d1, a00, a01, a10, a11, w_v, rows0, rows1,
                  idx0, idx1, z_v, acc,
                  ssem0, ssem1, dsem0, dsem1, asem0, asem1, bsem0, bsem1,
                  gsem0, gsem1):
        cid = lax.axis_index("c")
        sid = lax.axis_index("s")
        wid = sid * NC + cid

        zeros16 = jnp.zeros((16,), jnp.float32)
        for r in range(ZR):
            for c in range(D // 16):
                z_v[r, pl.ds(c * 16, 16)] = zeros16

        def zero_body(j, carry):
            pltpu.sync_copy(z_v, acc.at[pl.ds(sid * ROWS_PER_TILE + j * ZR, ZR)])
            return carry

        lax.fori_loop(0, ROWS_PER_TILE // ZR, zero_body, 0)
        plsc.subcore_barrier()

        pltpu.sync_copy(w_hbm, w_v)
        w0 = w_v[0]
        w1 = w_v[1]
        lane16 = lax.iota(jnp.int32, 16)

        def eload(c, s_ref, d_ref, a0_ref, a1_ref, ssem, dsem, asem, bsem):
            g = wid + NW * c
            off = jnp.minimum(g, FULL_CHUNKS - 1) * K
            pltpu.async_copy(ei_hbm.at[0, pl.ds(off, K)], s_ref, ssem)
            pltpu.async_copy(ei_hbm.at[1, pl.ds(off, K)], d_ref, dsem)
            pltpu.async_copy(a0_hbm.at[pl.ds(off, K)], a0_ref, asem)
            pltpu.async_copy(a1_hbm.at[pl.ds(off, K)], a1_ref, bsem)

        def ewait(s_ref, d_ref, a0_ref, a1_ref, ssem, dsem, asem, bsem):
            pltpu.make_async_copy(ei_hbm.at[0, pl.ds(0, K)], s_ref, ssem).wait()
            pltpu.make_async_copy(ei_hbm.at[1, pl.ds(0, K)], d_ref, dsem).wait()
            pltpu.make_async_copy(a0_hbm.at[pl.ds(0, K)], a0_ref, asem).wait()
            pltpu.make_async_copy(a1_hbm.at[pl.ds(0, K)], a1_ref, bsem).wait()

        def route(c, d_ref, a0_ref, a1_ref, idx_ref):
            scale = jnp.where((wid + NW * c) < FULL_CHUNKS, 1.0, 0.0)
            for v in range(K // 16):
                a0v = a0_ref[pl.ds(v * 16, 16)]
                a1v = a1_ref[pl.ds(v * 16, 16)]
                ew = w0 * a0v + w1 * a1v
                m = (ew * scale) != 0.0
                dummy_v = DUMMY + v * 16 + lane16
                idx_ref[pl.ds(v * 16, 16)] = jnp.where(
                    m, d_ref[pl.ds(v * 16, 16)], dummy_v)

        def gather(s_ref, rows_ref, sem):
            pltpu.async_copy(x_hbm.at[s_ref], rows_ref, sem)

        def gwait(rows_ref, sem):
            pltpu.make_async_copy(x_hbm.at[s0], rows_ref, sem).wait()

        eload(0, s0, d0, a00, a10, ssem0, dsem0, asem0, bsem0)
        eload(1, s1, d1, a01, a11, ssem1, dsem1, asem1, bsem1)
        ewait(s0, d0, a00, a10, ssem0, dsem0, asem0, bsem0)
        gather(s0, rows0, gsem0)

        def pair_body(j, carry):
            c0 = 2 * j
            ewait(s1, d1, a01, a11, ssem1, dsem1, asem1, bsem1)
            gather(s1, rows1, gsem1)
            route(c0, d0, a00, a10, idx0)
            gwait(rows0, gsem0)
            pltpu.sync_copy(rows0, acc.at[idx0], add=True)
            eload(c0 + 2, s0, d0, a00, a10, ssem0, dsem0, asem0, bsem0)
            ewait(s0, d0, a00, a10, ssem0, dsem0, asem0, bsem0)
            gather(s0, rows0, gsem0)
            route(c0 + 1, d1, a01, a11, idx1)
            gwait(rows1, gsem1)
            pltpu.sync_copy(rows1, acc.at[idx1], add=True)
            eload(c0 + 3, s1, d1, a01, a11, ssem1, dsem1, asem1, bsem1)
            return carry

        lax.fori_loop(0, (CHUNKS - 2) // 2, pair_body, 0)

        ewait(s1, d1, a01, a11, ssem1, dsem1, asem1, bsem1)
        gather(s1, rows1, gsem1)
        route(CHUNKS - 2, d0, a00, a10, idx0)
        gwait(rows0, gsem0)
        pltpu.sync_copy(rows0, acc.at[idx0], add=True)
        route(CHUNKS - 1, d1, a01, a11, idx1)
        gwait(rows1, gsem1)
        pltpu.sync_copy(rows1, acc.at[idx1], add=True)

        plsc.subcore_barrier()

        pltpu.sync_copy(
            acc.at[pl.ds(sid * ROWS_PER_TILE, ROWS_PER_TILE)],
            out_hbm.at[pl.ds(cid * NP + sid * ROWS_PER_TILE, ROWS_PER_TILE)],
        )

    return sc_kernel(ei, a0, a1, wvec, x)


def _tc_body(p0_ref, p1_ref, x_ref, wl_ref, wr_ref, b_ref, o_ref):
    agg = p0_ref[...] + p1_ref[...]
    dn = (((1,), (1,)), ((), ()))
    o_ref[...] = (
        lax.dot_general(agg, wl_ref[...], dn, preferred_element_type=jnp.float32)
        + lax.dot_general(x_ref[...], wr_ref[...], dn, preferred_element_type=jnp.float32)
        + b_ref[0:1, :]
    )


def _tc_dense(partials, x, W_l, W_r, b8):
    blk = 80
    grid = (N // blk,)
    return pl.pallas_call(
        _tc_body,
        grid=grid,
        in_specs=[
            pl.BlockSpec((blk, D), lambda i: (i, 0)),
            pl.BlockSpec((blk, D), lambda i: (i + NP // 80, 0)),
            pl.BlockSpec((blk, D), lambda i: (i, 0)),
            pl.BlockSpec((D, D), lambda i: (0, 0)),
            pl.BlockSpec((D, D), lambda i: (0, 0)),
            pl.BlockSpec((8, D), lambda i: (0, 0)),
        ],
        out_specs=pl.BlockSpec((blk, D), lambda i: (i, 0)),
        out_shape=jax.ShapeDtypeStruct((N, D), jnp.float32),
    )(partials, partials, x, W_l, W_r, b8)


def kernel(x, edge_index, A0, A1, ws_weights, W_l, b_l, W_r):
    ei = edge_index.astype(jnp.int32)
    wvec = jnp.stack([
        jnp.full((16,), ws_weights[0], jnp.float32),
        jnp.full((16,), ws_weights[1], jnp.float32),
    ])

    partials = _sc_scatter(ei, A0, A1, wvec, x)
    b8 = jnp.broadcast_to(b_l.reshape(1, D), (8, D))
    return _tc_dense(partials, x, W_l, W_r, b8)


# speedup vs baseline: 1.3051x; 1.3051x over previous
import functools

import jax
import jax.numpy as jnp
from jax import lax
from jax.experimental import pallas as pl
from jax.experimental.pallas import tpu as pltpu
from jax.experimental.pallas import tpu_sc as plsc

N = 10000
D = 128
E = 320000

NC = 2
NS = 16
NW = NC * NS
K = 128
FULL_CHUNKS = E // K
CHUNKS = 80
NP = 10240
ROWS_PER_TILE = NP // NS
DUMMY = N
ZR = 8


def _sc_scatter(ei, a0, a1, wvec, x):
    mesh = plsc.VectorSubcoreMesh(
        core_axis_name="c", subcore_axis_name="s", num_cores=NC, num_subcores=NS
    )

    @functools.partial(
        pl.kernel,
        out_type=jax.ShapeDtypeStruct((NC * NP, D), jnp.float32),
        mesh=mesh,
        scratch_types=[
            pltpu.VMEM((K,), jnp.int32),
            pltpu.VMEM((K,), jnp.int32),
            pltpu.VMEM((K,), jnp.int32),
            pltpu.VMEM((K,), jnp.int32),
            pltpu.VMEM((K,), jnp.float32),
            pltpu.VMEM((K,), jnp.float32),
            pltpu.VMEM((K,), jnp.float32),
            pltpu.VMEM((K,), jnp.float32),
            pltpu.VMEM((2, 16), jnp.float32),
            pltpu.VMEM((K, D), jnp.float32),
            pltpu.VMEM((K, D), jnp.float32),
            pltpu.VMEM((K,), jnp.int32),
            pltpu.VMEM((K,), jnp.int32),
            pltpu.VMEM((ZR, D), jnp.float32),
            pltpu.VMEM_SHARED((NP, D), jnp.float32),
            pltpu.SemaphoreType.DMA,
            pltpu.SemaphoreType.DMA,
            pltpu.SemaphoreType.DMA,
            pltpu.SemaphoreType.DMA,
            pltpu.SemaphoreType.DMA,
            pltpu.SemaphoreType.DMA,
            pltpu.SemaphoreType.DMA,
            pltpu.SemaphoreType.DMA,
            pltpu.SemaphoreType.DMA,
            pltpu.SemaphoreType.DMA,
        ],
    )
    def sc_kernel(ei_hbm, a0_hbm, a1_hbm, w_hbm, x_hbm, out_hbm,
                  s0, s1, d0, d1, a00, a01, a10, a11, w_v, rows0, rows1,
                  idx0, idx1, z_v, acc,
                  ssem0, ssem1, dsem0, dsem1, asem0, asem1, bsem0, bsem1,
                  gsem0, gsem1):
        cid = lax.axis_index("c")
        sid = lax.axis_index("s")
        wid = sid * NC + cid

        zeros16 = jnp.zeros((16,), jnp.float32)
        for r in range(ZR):
            for c in range(D // 16):
                z_v[r, pl.ds(c * 16, 16)] = zeros16

        def zero_body(j, carry):
            pltpu.sync_copy(z_v, acc.at[pl.ds(sid * ROWS_PER_TILE + j * ZR, ZR)])
            return carry

        lax.fori_loop(0, ROWS_PER_TILE // ZR, zero_body, 0)
        plsc.subcore_barrier()

        pltpu.sync_copy(w_hbm, w_v)
        w0 = w_v[0]
        w1 = w_v[1]
        lane16 = lax.iota(jnp.int32, 16)

        def eload(c, s_ref, d_ref, a0_ref, a1_ref, ssem, dsem, asem, bsem):
            g = wid + NW * c
            off = jnp.minimum(g, FULL_CHUNKS - 1) * K
            pltpu.async_copy(ei_hbm.at[0, pl.ds(off, K)], s_ref, ssem)
            pltpu.async_copy(ei_hbm.at[1, pl.ds(off, K)], d_ref, dsem)
            pltpu.async_copy(a0_hbm.at[pl.ds(off, K)], a0_ref, asem)
            pltpu.async_copy(a1_hbm.at[pl.ds(off, K)], a1_ref, bsem)

        def ewait(s_ref, d_ref, a0_ref, a1_ref, ssem, dsem, asem, bsem):
            pltpu.make_async_copy(ei_hbm.at[0, pl.ds(0, K)], s_ref, ssem).wait()
            pltpu.make_async_copy(ei_hbm.at[1, pl.ds(0, K)], d_ref, dsem).wait()
            pltpu.make_async_copy(a0_hbm.at[pl.ds(0, K)], a0_ref, asem).wait()
            pltpu.make_async_copy(a1_hbm.at[pl.ds(0, K)], a1_ref, bsem).wait()

        def route(c, d_ref, a0_ref, a1_ref, idx_ref):
            scale = jnp.where((wid + NW * c) < FULL_CHUNKS, 1.0, 0.0)
            for v in range(K // 16):
                a0v = a0_ref[pl.ds(v * 16, 16)]
                a1v = a1_ref[pl.ds(v * 16, 16)]
                ew = w0 * a0v + w1 * a1v
                m = (ew * scale) != 0.0
                dummy_v = DUMMY + v * 16 + lane16
                idx_ref[pl.ds(v * 16, 16)] = jnp.where(
                    m, d_ref[pl.ds(v * 16, 16)], dummy_v)

        def gather(s_ref, rows_ref, sem):
            pltpu.async_copy(x_hbm.at[s_ref], rows_ref, sem)

        def gwait(rows_ref, sem):
            pltpu.make_async_copy(x_hbm.at[s0], rows_ref, sem).wait()

        eload(0, s0, d0, a00, a10, ssem0, dsem0, asem0, bsem0)
        eload(1, s1, d1, a01, a11, ssem1, dsem1, asem1, bsem1)
        ewait(s0, d0, a00, a10, ssem0, dsem0, asem0, bsem0)
        gather(s0, rows0, gsem0)

        def pair_body(j, carry):
            c0 = 2 * j
            ewait(s1, d1, a01, a11, ssem1, dsem1, asem1, bsem1)
            gather(s1, rows1, gsem1)
            route(c0, d0, a00, a10, idx0)
            gwait(rows0, gsem0)
            pltpu.sync_copy(rows0, acc.at[idx0], add=True)
            eload(c0 + 2, s0, d0, a00, a10, ssem0, dsem0, asem0, bsem0)
            ewait(s0, d0, a00, a10, ssem0, dsem0, asem0, bsem0)
            gather(s0, rows0, gsem0)
            route(c0 + 1, d1, a01, a11, idx1)
            gwait(rows1, gsem1)
            pltpu.sync_copy(rows1, acc.at[idx1], add=True)
            eload(c0 + 3, s1, d1, a01, a11, ssem1, dsem1, asem1, bsem1)
            return carry

        lax.fori_loop(0, (CHUNKS - 2) // 2, pair_body, 0)

        ewait(s1, d1, a01, a11, ssem1, dsem1, asem1, bsem1)
        gather(s1, rows1, gsem1)
        route(CHUNKS - 2, d0, a00, a10, idx0)
        gwait(rows0, gsem0)
        pltpu.sync_copy(rows0, acc.at[idx0], add=True)
        route(CHUNKS - 1, d1, a01, a11, idx1)
        gwait(rows1, gsem1)
        pltpu.sync_copy(rows1, acc.at[idx1], add=True)

        plsc.subcore_barrier()

        pltpu.sync_copy(
            acc.at[pl.ds(sid * ROWS_PER_TILE, ROWS_PER_TILE)],
            out_hbm.at[pl.ds(cid * NP + sid * ROWS_PER_TILE, ROWS_PER_TILE)],
        )

    return sc_kernel(ei, a0, a1, wvec, x)


def _tc_body(p0_ref, p1_ref, x_ref, wl_ref, wr_ref, b_ref, o_ref):
    agg = p0_ref[...] + p1_ref[...]
    dn = (((1,), (1,)), ((), ()))
    o_ref[...] = (
        lax.dot_general(agg, wl_ref[...], dn, preferred_element_type=jnp.float32)
        + lax.dot_general(x_ref[...], wr_ref[...], dn, preferred_element_type=jnp.float32)
        + b_ref[0:1, :]
    )


def _tc_dense(p0, p1, x, W_l, W_r, b8):
    blk = 1000
    grid = (N // blk,)
    return pl.pallas_call(
        _tc_body,
        grid=grid,
        in_specs=[
            pl.BlockSpec((blk, D), lambda i: (i, 0)),
            pl.BlockSpec((blk, D), lambda i: (i, 0)),
            pl.BlockSpec((blk, D), lambda i: (i, 0)),
            pl.BlockSpec((D, D), lambda i: (0, 0)),
            pl.BlockSpec((D, D), lambda i: (0, 0)),
            pl.BlockSpec((8, D), lambda i: (0, 0)),
        ],
        out_specs=pl.BlockSpec((blk, D), lambda i: (i, 0)),
        out_shape=jax.ShapeDtypeStruct((N, D), jnp.float32),
    )(p0, p1, x, W_l, W_r, b8)


def kernel(x, edge_index, A0, A1, ws_weights, W_l, b_l, W_r):
    ei = edge_index.astype(jnp.int32)
    wvec = jnp.stack([
        jnp.full((16,), ws_weights[0], jnp.float32),
        jnp.full((16,), ws_weights[1], jnp.float32),
    ])

    partials = _sc_scatter(ei, A0, A1, wvec, x)
    p0 = partials[:N]
    p1 = partials[NP:NP + N]
    b8 = jnp.broadcast_to(b_l.reshape(1, D), (8, D))
    return _tc_dense(p0, p1, x, W_l, W_r, b8)
